# Initial kernel scaffold; baseline (speedup 1.0000x reference)
import jax
import jax.numpy as jnp
from jax.experimental import pallas as pl


def kernel(embeddings, labels, bonafide_centers):
    raise NotImplementedError("write your pallas kernel here")


# speedup vs baseline: 1.2156x; 1.2156x over previous
import functools

import jax
import jax.numpy as jnp
from jax.experimental import pallas as pl
from jax.experimental.pallas import tpu as pltpu

B = 16384
K = 1024
D = 512
ALPHA = 1.0

BM = 2048
NB = B // BM


def _loss_kernel(emb_ref, lab_ref, cen_ref, out_ref, acc_ref):
    i = pl.program_id(0)

    @pl.when(i == 0)
    def _init():
        acc_ref[0] = 0.0
        acc_ref[1] = 0.0
        acc_ref[2] = 0.0
        acc_ref[3] = 0.0

    c = cen_ref[...]
    cn = c / jnp.maximum(jnp.sqrt(jnp.sum(c * c, axis=1, keepdims=True)), 1e-12)
    b2 = jnp.sum(cn * cn, axis=1, keepdims=True).T

    e = emb_ref[...]
    en = e / jnp.maximum(jnp.sqrt(jnp.sum(e * e, axis=1, keepdims=True)), 1e-12)
    a2 = jnp.sum(en * en, axis=1, keepdims=True)

    dot = jax.lax.dot_general(
        en, cn, (((1,), (1,)), ((), ())),
        preferred_element_type=jnp.float32,
    )
    d2 = a2 + b2 - 2.0 * dot
    d = jnp.sqrt(jnp.maximum(d2, 1e-12))
    min_d2 = jnp.min(d, axis=1, keepdims=True) ** 2

    lab = lab_ref[...]
    bona = lab == 0.0
    spoof = lab == 1.0
    acc_ref[0] += jnp.sum(jnp.where(bona, min_d2, 0.0))
    acc_ref[1] += jnp.sum(jnp.where(spoof, min_d2, 0.0))
    acc_ref[2] += jnp.sum(jnp.where(bona, 1.0, 0.0))
    acc_ref[3] += jnp.sum(jnp.where(spoof, 1.0, 0.0))

    @pl.when(i == NB - 1)
    def _finalize():
        n_bona = acc_ref[2]
        n_spoof = acc_ref[3]
        bona_loss = acc_ref[0] / jnp.maximum(n_bona, 1.0)
        spoof_loss = -ALPHA * (acc_ref[1] / jnp.maximum(n_spoof, 1.0))
        total = (jnp.where(n_bona > 0.0, bona_loss, 0.0)
                 + jnp.where(n_spoof > 0.0, spoof_loss, 0.0))
        out_ref[0, 0] = total


@functools.partial(jax.jit, static_argnames=("interpret",))
def kernel(embeddings, labels, bonafide_centers, interpret=False):
    lab = labels.astype(jnp.float32).reshape(B, 1)
    out = pl.pallas_call(
        _loss_kernel,
        grid=(NB,),
        in_specs=[
            pl.BlockSpec((BM, D), lambda i: (i, 0)),
            pl.BlockSpec((BM, 1), lambda i: (i, 0)),
            pl.BlockSpec((K, D), lambda i: (0, 0)),
        ],
        out_specs=pl.BlockSpec(memory_space=pltpu.SMEM),
        out_shape=jax.ShapeDtypeStruct((1, 1), jnp.float32),
        scratch_shapes=[pltpu.SMEM((4,), jnp.float32)],
        interpret=interpret,
    )(embeddings, lab, bonafide_centers)
    return out[0, 0]


# speedup vs baseline: 1.9745x; 1.6243x over previous
import functools

import jax
import jax.numpy as jnp
from jax.experimental import pallas as pl
from jax.experimental.pallas import tpu as pltpu

B = 16384
K = 1024
D = 512
ALPHA = 1.0

BM = 2048
NB = B // BM


def _loss_kernel(emb_ref, lab_ref, cent_ref, out_ref, acc_ref, cn_ref):
    i = pl.program_id(0)

    @pl.when(i == 0)
    def _init():
        acc_ref[0] = 0.0
        acc_ref[1] = 0.0
        acc_ref[2] = 0.0
        acc_ref[3] = 0.0
        ct = cent_ref[...]
        cs = jnp.sum(ct * ct, axis=0, keepdims=True)
        inv = -2.0 / jnp.maximum(jnp.sqrt(cs), 1e-12)
        cn_ref[...] = ct * inv

    e = emb_ref[...]
    es = jnp.sum(e * e, axis=1, keepdims=True)
    en = e * (1.0 / jnp.maximum(jnp.sqrt(es), 1e-12))

    dot = jax.lax.dot_general(
        en, cn_ref[...], (((1,), (0,)), ((), ())),
        preferred_element_type=jnp.float32,
    )
    min_d2 = jnp.maximum(2.0 + jnp.min(dot, axis=1, keepdims=True), 1e-12)

    lab = lab_ref[...]
    bona = lab == 0.0
    spoof = lab == 1.0
    acc_ref[0] += jnp.sum(jnp.where(bona, min_d2, 0.0))
    acc_ref[1] += jnp.sum(jnp.where(spoof, min_d2, 0.0))
    acc_ref[2] += jnp.sum(jnp.where(bona, 1.0, 0.0))
    acc_ref[3] += jnp.sum(jnp.where(spoof, 1.0, 0.0))

    @pl.when(i == NB - 1)
    def _finalize():
        n_bona = acc_ref[2]
        n_spoof = acc_ref[3]
        bona_loss = acc_ref[0] / jnp.maximum(n_bona, 1.0)
        spoof_loss = -ALPHA * (acc_ref[1] / jnp.maximum(n_spoof, 1.0))
        total = (jnp.where(n_bona > 0.0, bona_loss, 0.0)
                 + jnp.where(n_spoof > 0.0, spoof_loss, 0.0))
        out_ref[0, 0] = total


@functools.partial(jax.jit, static_argnames=("interpret",))
def kernel(embeddings, labels, bonafide_centers, interpret=False):
    lab = labels.astype(jnp.float32).reshape(B, 1)
    cent_t = bonafide_centers.T
    out = pl.pallas_call(
        _loss_kernel,
        grid=(NB,),
        in_specs=[
            pl.BlockSpec((BM, D), lambda i: (i, 0)),
            pl.BlockSpec((BM, 1), lambda i: (i, 0)),
            pl.BlockSpec((D, K), lambda i: (0, 0)),
        ],
        out_specs=pl.BlockSpec(memory_space=pltpu.SMEM),
        out_shape=jax.ShapeDtypeStruct((1, 1), jnp.float32),
        scratch_shapes=[pltpu.SMEM((4,), jnp.float32),
                        pltpu.VMEM((D, K), jnp.float32)],
        interpret=interpret,
    )(embeddings, lab, cent_t)
    return out[0, 0]
